# Initial kernel scaffold; baseline (speedup 1.0000x reference)
import jax
import jax.numpy as jnp
from jax.experimental import pallas as pl


def kernel(input_tokens, input_boxes, embed_boxes, token_table, bbox_tables):
    raise NotImplementedError("write your pallas kernel here")


# speedup vs baseline: 5.9860x; 5.9860x over previous
import functools

import jax
import jax.numpy as jnp
from jax import lax
from jax.experimental import pallas as pl
from jax.experimental.pallas import tpu as pltpu
from jax.experimental.pallas import tpu_sc as plsc

VOCAB = 100000
HIDDEN = 128
BBOX_VOCAB = 1004
BBOX_DIM = 64
B, S = 4096, 50
N = B * S

NC, NS, L = 2, 16, 16
NW = NC * NS
TOK_PER_W = N // NW
CHUNK = 80
NCHUNK = TOK_PER_W // CHUNK
ZROW = 6 * BBOX_VOCAB


def _body(tok_hbm, boxt_hbm, ttab_hbm, btab_hbm, out_hbm,
          tidx_v, bidx_v, bidx2_v, trows_v, brows_v, sem, osem):
    wid = lax.axis_index("s") * NC + lax.axis_index("c")
    wbase = wid * TOK_PER_W

    @pl.loop(0, NCHUNK)
    def _chunk(c):
        base = wbase + c * CHUNK

        pltpu.sync_copy(tok_hbm.at[pl.ds(base, CHUNK)], tidx_v)
        for i in range(6):
            pltpu.sync_copy(boxt_hbm.at[pl.ds(i * N + base, CHUNK)],
                            bidx_v.at[i])

        for g in range(CHUNK // L):
            sl = pl.ds(g * L, L)
            box0 = bidx_v[0, sl]
            keep = box0 < 1001
            for i in range(6):
                raw = bidx_v[i, sl] if i else box0
                bidx2_v[i, sl] = jnp.where(keep, raw + i * BBOX_VOCAB, ZROW)

        cps = [pltpu.async_copy(ttab_hbm.at[tidx_v], trows_v, sem)]
        for i in range(6):
            cps.append(
                pltpu.async_copy(btab_hbm.at[bidx2_v.at[i]], brows_v.at[i], sem))
        for cp in cps:
            cp.wait()

        @pl.loop(0, CHUNK)
        def _tok(t):
            for j in range(BBOX_DIM // L):
                sl = pl.ds(j * L, L)
                acc = trows_v[t, pl.ds(BBOX_DIM + j * L, L)]
                for i in range(6):
                    acc = acc + brows_v[i, t, sl]
                trows_v[t, pl.ds(BBOX_DIM + j * L, L)] = acc

        pltpu.async_copy(trows_v, out_hbm.at[pl.ds(base, CHUNK)], osem).wait()


@jax.jit
def _run(tok_flat, boxes_t, token_table, btab):
    kern = pl.kernel(
        _body,
        out_type=jax.ShapeDtypeStruct((N, HIDDEN), jnp.float32),
        mesh=plsc.VectorSubcoreMesh(
            core_axis_name="c", subcore_axis_name="s",
            num_cores=NC, num_subcores=NS),
        scratch_types=[
            pltpu.VMEM((CHUNK,), jnp.int32),
            pltpu.VMEM((6, CHUNK), jnp.int32),
            pltpu.VMEM((6, CHUNK), jnp.int32),
            pltpu.VMEM((CHUNK, HIDDEN), jnp.float32),
            pltpu.VMEM((6, CHUNK, BBOX_DIM), jnp.float32),
            pltpu.SemaphoreType.DMA,
            pltpu.SemaphoreType.DMA,
        ],
        compiler_params=pltpu.CompilerParams(use_tc_tiling_on_sc=False),
    )
    return kern(tok_flat, boxes_t, token_table, btab)


def kernel(input_tokens, input_boxes, embed_boxes, token_table, bbox_tables):
    tok_flat = input_tokens.astype(jnp.int32).reshape(N)
    boxes_t = input_boxes.astype(jnp.int32).reshape(N, 6).T.reshape(6 * N)
    btab = jnp.concatenate(
        [bbox_tables.reshape(6 * BBOX_VOCAB, BBOX_DIM),
         jnp.zeros((8, BBOX_DIM), jnp.float32)])
    out = _run(tok_flat, boxes_t, token_table, btab)
    return out.reshape(B, S, HIDDEN)


# speedup vs baseline: 7.4974x; 1.2525x over previous
import jax
import jax.numpy as jnp
from jax import lax
from jax.experimental import pallas as pl
from jax.experimental.pallas import tpu as pltpu
from jax.experimental.pallas import tpu_sc as plsc

VOCAB = 100000
HIDDEN = 128
BBOX_VOCAB = 1004
BBOX_DIM = 64
B, S = 4096, 50
N = B * S

NC, NS, L = 2, 16, 16
NW = NC * NS
TOK_PER_W = N // NW
CHUNK = 80
NCHUNK = TOK_PER_W // CHUNK
ZROW = 6 * BBOX_VOCAB


def _body(tok_hbm, boxt_hbm, ttab_hbm, btab_hbm, out_hbm,
          tidx_v, bidx_v, bidx2_v, trows_v, brows_v, gsem, osem):
    wid = lax.axis_index("s") * NC + lax.axis_index("c")
    wbase = wid * TOK_PER_W

    def stage(c, tb, bb, first=False):
        base = wbase + c * CHUNK
        if not first:
            pltpu.make_async_copy(
                trows_v.at[tb], out_hbm.at[pl.ds(base, CHUNK)],
                osem.at[tb]).wait()
        pltpu.sync_copy(tok_hbm.at[pl.ds(base, CHUNK)], tidx_v.at[bb])
        for i in range(6):
            pltpu.sync_copy(boxt_hbm.at[pl.ds(i * N + base, CHUNK)],
                            bidx_v.at[bb, i])
        for g in range(CHUNK // L):
            sl = pl.ds(g * L, L)
            box0 = bidx_v[bb, 0, sl]
            keep = box0 < 1001
            for i in range(6):
                raw = bidx_v[bb, i, sl] if i else box0
                bidx2_v[bb, i, sl] = jnp.where(keep, raw + i * BBOX_VOCAB, ZROW)
        pltpu.async_copy(ttab_hbm.at[tidx_v.at[bb]], trows_v.at[tb],
                         gsem.at[bb])
        for i in range(6):
            pltpu.async_copy(btab_hbm.at[bidx2_v.at[bb, i]],
                             brows_v.at[bb, i], gsem.at[bb])

    def finish(c, tb, bb):
        base = wbase + c * CHUNK
        pltpu.make_async_copy(ttab_hbm.at[tidx_v.at[bb]], trows_v.at[tb],
                              gsem.at[bb]).wait()
        for i in range(6):
            pltpu.make_async_copy(btab_hbm.at[bidx2_v.at[bb, i]],
                                  brows_v.at[bb, i], gsem.at[bb]).wait()

        @pl.loop(0, CHUNK)
        def _tok(t):
            for j in range(BBOX_DIM // L):
                acc = trows_v[tb, t, pl.ds(BBOX_DIM + j * L, L)]
                for i in range(6):
                    acc = acc + brows_v[bb, i, t, pl.ds(j * L, L)]
                trows_v[tb, t, pl.ds(BBOX_DIM + j * L, L)] = acc

        pltpu.async_copy(trows_v.at[tb], out_hbm.at[pl.ds(base, CHUNK)],
                         osem.at[tb])

    stage(0, 0, 0, first=True)
    stage(1, 1, 1, first=True)
    finish(0, 0, 0)
    stage(2, 2, 0, first=True)
    finish(1, 1, 1)
    stage(3, 3, 1, first=True)

    @pl.loop(1, NCHUNK // 4)
    def _grp(k):
        c = 4 * k
        finish(c - 2, 2, 0)
        stage(c, 0, 0)
        finish(c - 1, 3, 1)
        stage(c + 1, 1, 1)
        finish(c, 0, 0)
        stage(c + 2, 2, 0)
        finish(c + 1, 1, 1)
        stage(c + 3, 3, 1)

    finish(NCHUNK - 2, 2, 0)
    finish(NCHUNK - 1, 3, 1)
    for tb in range(4):
        pltpu.make_async_copy(trows_v.at[tb], out_hbm.at[pl.ds(wbase, CHUNK)],
                              osem.at[tb]).wait()


@jax.jit
def _run(tok_flat, boxes_t, token_table, btab):
    kern = pl.kernel(
        _body,
        out_type=jax.ShapeDtypeStruct((N, HIDDEN), jnp.float32),
        mesh=plsc.VectorSubcoreMesh(
            core_axis_name="c", subcore_axis_name="s",
            num_cores=NC, num_subcores=NS),
        scratch_types=[
            pltpu.VMEM((2, CHUNK), jnp.int32),
            pltpu.VMEM((2, 6, CHUNK), jnp.int32),
            pltpu.VMEM((2, 6, CHUNK), jnp.int32),
            pltpu.VMEM((4, CHUNK, HIDDEN), jnp.float32),
            pltpu.VMEM((2, 6, CHUNK, BBOX_DIM), jnp.float32),
            pltpu.SemaphoreType.DMA((2,)),
            pltpu.SemaphoreType.DMA((4,)),
        ],
        compiler_params=pltpu.CompilerParams(use_tc_tiling_on_sc=False),
    )
    return kern(tok_flat, boxes_t, token_table, btab)


def kernel(input_tokens, input_boxes, embed_boxes, token_table, bbox_tables):
    tok_flat = input_tokens.astype(jnp.int32).reshape(N)
    boxes_t = input_boxes.astype(jnp.int32).reshape(N, 6).T.reshape(6 * N)
    btab = jnp.concatenate(
        [bbox_tables.reshape(6 * BBOX_VOCAB, BBOX_DIM),
         jnp.zeros((8, BBOX_DIM), jnp.float32)])
    out = _run(tok_flat, boxes_t, token_table, btab)
    return out.reshape(B, S, HIDDEN)
